# Initial kernel scaffold; baseline (speedup 1.0000x reference)
import jax
import jax.numpy as jnp
from jax.experimental import pallas as pl


def kernel(h, e, edge_index, graph_ids, W_embed, b_embed, W0, b0, W1, b1, W2, b2, p_pos, p_neg, W_fc):
    raise NotImplementedError("write your pallas kernel here")


# speedup vs baseline: 8.1605x; 8.1605x over previous
import functools

import jax
import jax.numpy as jnp
from jax import lax
from jax.experimental import pallas as pl
from jax.experimental.pallas import tpu as pltpu
from jax.experimental.pallas import tpu_sc as plsc

N = 10000
E = 320000
D = 128
G = 50
NPG = 200
NPROT = 10

NC = 2
NS = 16
NW = NC * NS
EPW = E // NW
C = 40
NCH = EPW // C
NP = 10240
RPS = NP // NS
RCH = 40
NZ = RPS // RCH
DEGW = 16


def _sc_mesh():
    return plsc.VectorSubcoreMesh(
        core_axis_name="c", subcore_axis_name="s",
        num_cores=NC, num_subcores=NS)


def _zero_fill(buf, nrows, width):
    zero16 = jnp.zeros((16,), jnp.float32)

    def _f(i, carry):
        for jj in range(width // 16):
            buf[i, pl.ds(jj * 16, 16)] = zero16
        return carry
    lax.fori_loop(0, nrows, _f, 0)


def _agg_body(x_hbm, src_hbm, dst_hbm, out,
              srcv, dstv, rows, accum, sem0, sem1):
    cid = lax.axis_index("c")
    sid = lax.axis_index("s")
    w = cid * NS + sid
    base = sid * RPS

    _zero_fill(rows.at[0], RCH, D)
    for z in range(NZ):
        pltpu.sync_copy(rows.at[0], accum.at[pl.ds(base + z * RCH, RCH)])

    pltpu.sync_copy(src_hbm.at[w], srcv)
    pltpu.sync_copy(dst_hbm.at[w], dstv)

    plsc.subcore_barrier()

    pltpu.async_copy(x_hbm.at[srcv.at[0]], rows.at[0], sem0)

    def _pair(t, carry):
        j0 = 2 * t
        pltpu.async_copy(x_hbm.at[srcv.at[j0 + 1]], rows.at[1], sem1)
        pltpu.make_async_copy(x_hbm.at[srcv.at[j0]], rows.at[0], sem0).wait()
        pltpu.sync_copy(rows.at[0], accum.at[dstv.at[j0]], add=True)

        @pl.when(t < NCH // 2 - 1)
        def _():
            pltpu.async_copy(x_hbm.at[srcv.at[j0 + 2]], rows.at[0], sem0)

        pltpu.make_async_copy(
            x_hbm.at[srcv.at[j0 + 1]], rows.at[1], sem1).wait()
        pltpu.sync_copy(rows.at[1], accum.at[dstv.at[j0 + 1]], add=True)
        return carry
    lax.fori_loop(0, NCH // 2, _pair, 0)

    plsc.subcore_barrier()

    sl = pl.ds(base, RPS)
    pltpu.sync_copy(accum.at[sl], out.at[cid, sl])


def _deg_body(dst_hbm, deg, dstv, onesv, stage, degacc, sem0):
    cid = lax.axis_index("c")
    sid = lax.axis_index("s")
    w = cid * NS + sid
    base = sid * RPS

    _zero_fill(stage, RCH, DEGW)
    for z in range(NZ):
        pltpu.sync_copy(stage, degacc.at[pl.ds(base + z * RCH, RCH)])
    ones16 = jnp.ones((16,), jnp.float32)

    def _f(i, carry):
        onesv[i, pl.ds(0, DEGW)] = ones16
        return carry
    lax.fori_loop(0, C, _f, 0)
    pltpu.sync_copy(dst_hbm.at[w], dstv)

    plsc.subcore_barrier()

    def _chunk(j, carry):
        pltpu.sync_copy(onesv, degacc.at[dstv.at[j]], add=True)
        return carry
    lax.fori_loop(0, NCH, _chunk, 0)

    plsc.subcore_barrier()

    sl = pl.ds(base, RPS)
    pltpu.sync_copy(degacc.at[sl], deg.at[cid, sl])


@functools.lru_cache(maxsize=None)
def _make_agg():
    return pl.kernel(
        _agg_body,
        out_type=jax.ShapeDtypeStruct((NC, NP, D), jnp.float32),
        mesh=_sc_mesh(),
        scratch_types=[
            pltpu.VMEM((NCH, C), jnp.int32),
            pltpu.VMEM((NCH, C), jnp.int32),
            pltpu.VMEM((2, C, D), jnp.float32),
            pltpu.VMEM_SHARED((NP, D), jnp.float32),
            pltpu.SemaphoreType.DMA,
            pltpu.SemaphoreType.DMA,
        ],
        compiler_params=pltpu.CompilerParams(use_tc_tiling_on_sc=False),
    )


@functools.lru_cache(maxsize=None)
def _make_deg():
    return pl.kernel(
        _deg_body,
        out_type=jax.ShapeDtypeStruct((NC, NP, DEGW), jnp.float32),
        mesh=_sc_mesh(),
        scratch_types=[
            pltpu.VMEM((NCH, C), jnp.int32),
            pltpu.VMEM((C, DEGW), jnp.float32),
            pltpu.VMEM((RCH, DEGW), jnp.float32),
            pltpu.VMEM_SHARED((NP, DEGW), jnp.float32),
            pltpu.SemaphoreType.DMA,
        ],
        compiler_params=pltpu.CompilerParams(use_tc_tiling_on_sc=False),
    )


RB = 2000


def _embed_body(h_ref, w_ref, b_ref, o_ref):
    o_ref[...] = (
        jnp.dot(h_ref[...], w_ref[...], preferred_element_type=jnp.float32)
        + b_ref[...]
    )


def _embed(h, W, b):
    return pl.pallas_call(
        _embed_body,
        grid=(N // RB,),
        in_specs=[
            pl.BlockSpec((RB, D), lambda i: (i, 0)),
            pl.BlockSpec((D, D), lambda i: (0, 0)),
            pl.BlockSpec((1, D), lambda i: (0, 0)),
        ],
        out_specs=pl.BlockSpec((RB, D), lambda i: (i, 0)),
        out_shape=jax.ShapeDtypeStruct((N, D), jnp.float32),
    )(h, W, b.reshape(1, D))


def _sage_update(x, p0, p1, invdeg, wt, wb, b):
    hn = (p0 + p1) * invdeg
    bundle = (
        jnp.dot(x, wt, preferred_element_type=jnp.float32)
        + jnp.dot(hn, wb, preferred_element_type=jnp.float32)
        + b
    )
    nrm = jnp.sqrt(jnp.sum(bundle * bundle, axis=1, keepdims=True))
    return x + jnp.maximum(bundle / jnp.maximum(nrm, 1e-12), 0.0)


def _layer_body(x_ref, p0_ref, p1_ref, iv_ref, wt_ref, wb_ref, b_ref, o_ref):
    o_ref[...] = _sage_update(
        x_ref[...], p0_ref[0], p1_ref[0], iv_ref[...][:, :1],
        wt_ref[...], wb_ref[...], b_ref[...])


def _layer(x, part, invdeg, W, b):
    return pl.pallas_call(
        _layer_body,
        grid=(N // RB,),
        in_specs=[
            pl.BlockSpec((RB, D), lambda i: (i, 0)),
            pl.BlockSpec((1, RB, D), lambda i: (0, i, 0)),
            pl.BlockSpec((1, RB, D), lambda i: (1, i, 0)),
            pl.BlockSpec((RB, DEGW), lambda i: (i, 0)),
            pl.BlockSpec((D, D), lambda i: (0, 0)),
            pl.BlockSpec((D, D), lambda i: (0, 0)),
            pl.BlockSpec((1, D), lambda i: (0, 0)),
        ],
        out_specs=pl.BlockSpec((RB, D), lambda i: (i, 0)),
        out_shape=jax.ShapeDtypeStruct((N, D), jnp.float32),
    )(x, part, part, invdeg, W[:D], W[D:], b.reshape(1, D))


def _final_body(x_ref, p0_ref, p1_ref, iv_ref, wt_ref, wb_ref, b_ref,
                prot_ref, wfc_ref, o_ref):
    x3 = _sage_update(
        x_ref[...], p0_ref[0], p1_ref[0], iv_ref[...][:, :1],
        wt_ref[...], wb_ref[...], b_ref[...])
    P = prot_ref[...]
    wfc = wfc_ref[...]
    y = jnp.zeros((), jnp.float32)
    for k in range(NPROT):
        diff = x3 - P[k:k + 1, :]
        d2 = jnp.sum(diff * diff, axis=1)
        sim = jnp.log((d2 + 1.0) / (d2 + 1e-12))
        y = y + jnp.max(sim) * wfc[0, k]
    o_ref[...] = jnp.full((1, 8, D), 1.0 / (1.0 + jnp.exp(-y)), jnp.float32)


def _final(x, part, invdeg, W, b, prot, wfc):
    return pl.pallas_call(
        _final_body,
        grid=(G,),
        in_specs=[
            pl.BlockSpec((NPG, D), lambda i: (i, 0)),
            pl.BlockSpec((1, NPG, D), lambda i: (0, i, 0)),
            pl.BlockSpec((1, NPG, D), lambda i: (1, i, 0)),
            pl.BlockSpec((NPG, DEGW), lambda i: (i, 0)),
            pl.BlockSpec((D, D), lambda i: (0, 0)),
            pl.BlockSpec((D, D), lambda i: (0, 0)),
            pl.BlockSpec((1, D), lambda i: (0, 0)),
            pl.BlockSpec((16, D), lambda i: (0, 0)),
            pl.BlockSpec((8, D), lambda i: (0, 0)),
        ],
        out_specs=pl.BlockSpec((1, 8, D), lambda i: (i, 0, 0)),
        out_shape=jax.ShapeDtypeStruct((G, 8, D), jnp.float32),
    )(x, part, part, invdeg, W[:D], W[D:], b.reshape(1, D), prot, wfc)


def kernel(h, e, edge_index, graph_ids, W_embed, b_embed,
           W0, b0, W1, b1, W2, b2, p_pos, p_neg, W_fc):
    src3 = edge_index[0].reshape(NW, NCH, C)
    dst3 = edge_index[1].reshape(NW, NCH, C)

    x0 = _embed(h, W_embed, b_embed)
    dg = _make_deg()(dst3)
    p0 = _make_agg()(x0, src3, dst3)
    invdeg = (1.0 / jnp.maximum(dg[0, :, :1] + dg[1, :, :1], 1.0)) * jnp.ones(
        (1, DEGW), jnp.float32)

    x1 = _layer(x0, p0, invdeg, W0, b0)
    p1 = _make_agg()(x1, src3, dst3)
    x2 = _layer(x1, p1, invdeg, W1, b1)
    p2 = _make_agg()(x2, src3, dst3)

    prot = jnp.zeros((16, D), jnp.float32).at[:NPROT].set(
        jnp.concatenate([p_pos, p_neg], axis=0))
    wfc = jnp.zeros((8, D), jnp.float32).at[0, :NPROT].set(W_fc[0])
    y = _final(x2, p2, invdeg, W2, b2, prot, wfc)
    return y[:, 0, 0]


# speedup vs baseline: 10.2752x; 1.2591x over previous
import functools

import jax
import jax.numpy as jnp
from jax import lax
from jax.experimental import pallas as pl
from jax.experimental.pallas import tpu as pltpu
from jax.experimental.pallas import tpu_sc as plsc

N = 10000
E = 320000
D = 128
G = 50
NPG = 200
NPROT = 10

NC = 2
NS = 16
NW = NC * NS
EPW = E // NW
C = 80
NCH = EPW // C
NP = 10240
RPS = NP // NS
RCH = 80
NZ = RPS // RCH
DEGW = 16
IDXSHIFT = 14


def _sc_mesh():
    return plsc.VectorSubcoreMesh(
        core_axis_name="c", subcore_axis_name="s",
        num_cores=NC, num_subcores=NS)


def _zero_fill(buf, nrows, width):
    zero16 = jnp.zeros((16,), jnp.float32)

    def _f(i, carry):
        for jj in range(width // 16):
            buf[i, pl.ds(jj * 16, 16)] = zero16
        return carry
    lax.fori_loop(0, nrows, _f, 0)


def _agg_body(x_hbm, packed_hbm, out,
              packv, srcv, dstv, rows, accum, sem0, sem1):
    cid = lax.axis_index("c")
    sid = lax.axis_index("s")
    w = cid * NS + sid
    base = sid * RPS

    _zero_fill(rows.at[0], RCH, D)
    for z in range(NZ):
        pltpu.sync_copy(rows.at[0], accum.at[pl.ds(base + z * RCH, RCH)])

    pltpu.sync_copy(packed_hbm.at[w], packv)

    def _unpack(j, b):
        for q in range(C // 16):
            v = packv[j, pl.ds(q * 16, 16)]
            srcv[b, pl.ds(q * 16, 16)] = v & ((1 << IDXSHIFT) - 1)
            dstv[b, pl.ds(q * 16, 16)] = v >> IDXSHIFT

    plsc.subcore_barrier()

    _unpack(0, 0)
    pltpu.async_copy(x_hbm.at[srcv.at[0]], rows.at[0], sem0)

    def _pair(t, carry):
        j0 = 2 * t
        _unpack(j0 + 1, 1)
        pltpu.async_copy(x_hbm.at[srcv.at[1]], rows.at[1], sem1)
        pltpu.make_async_copy(x_hbm.at[srcv.at[0]], rows.at[0], sem0).wait()
        pltpu.sync_copy(rows.at[0], accum.at[dstv.at[0]], add=True)
        _unpack(j0 + 2, 0)
        pltpu.async_copy(x_hbm.at[srcv.at[0]], rows.at[0], sem0)
        pltpu.make_async_copy(x_hbm.at[srcv.at[1]], rows.at[1], sem1).wait()
        pltpu.sync_copy(rows.at[1], accum.at[dstv.at[1]], add=True)
        return carry
    lax.fori_loop(0, (NCH - 1) // 2, _pair, 0)

    pltpu.make_async_copy(x_hbm.at[srcv.at[0]], rows.at[0], sem0).wait()
    pltpu.sync_copy(rows.at[0], accum.at[dstv.at[0]], add=True)

    plsc.subcore_barrier()

    sl = pl.ds(base, RPS)
    pltpu.sync_copy(accum.at[sl], out.at[cid, sl])


def _deg_body(packed_hbm, deg, packv, dstv, onesv, stage, degacc, sem0):
    cid = lax.axis_index("c")
    sid = lax.axis_index("s")
    w = cid * NS + sid
    base = sid * RPS

    _zero_fill(stage, RCH, DEGW)
    for z in range(NZ):
        pltpu.sync_copy(stage, degacc.at[pl.ds(base + z * RCH, RCH)])
    ones16 = jnp.ones((16,), jnp.float32)

    def _f(i, carry):
        onesv[i, pl.ds(0, DEGW)] = ones16
        return carry
    lax.fori_loop(0, C, _f, 0)
    pltpu.sync_copy(packed_hbm.at[w], packv)

    plsc.subcore_barrier()

    def _chunk(j, carry):
        for q in range(C // 16):
            dstv[0, pl.ds(q * 16, 16)] = (
                packv[j, pl.ds(q * 16, 16)] >> IDXSHIFT)
        pltpu.sync_copy(onesv, degacc.at[dstv.at[0]], add=True)
        return carry
    lax.fori_loop(0, NCH, _chunk, 0)

    plsc.subcore_barrier()

    sl = pl.ds(base, RPS)
    pltpu.sync_copy(degacc.at[sl], deg.at[cid, sl])


@functools.lru_cache(maxsize=None)
def _make_agg():
    return pl.kernel(
        _agg_body,
        out_type=jax.ShapeDtypeStruct((NC, NP, D), jnp.float32),
        mesh=_sc_mesh(),
        scratch_types=[
            pltpu.VMEM((NCH, C), jnp.int32),
            pltpu.VMEM((2, C), jnp.int32),
            pltpu.VMEM((2, C), jnp.int32),
            pltpu.VMEM((2, C, D), jnp.float32),
            pltpu.VMEM_SHARED((NP, D), jnp.float32),
            pltpu.SemaphoreType.DMA,
            pltpu.SemaphoreType.DMA,
        ],
        compiler_params=pltpu.CompilerParams(use_tc_tiling_on_sc=False),
    )


@functools.lru_cache(maxsize=None)
def _make_deg():
    return pl.kernel(
        _deg_body,
        out_type=jax.ShapeDtypeStruct((NC, NP, DEGW), jnp.float32),
        mesh=_sc_mesh(),
        scratch_types=[
            pltpu.VMEM((NCH, C), jnp.int32),
            pltpu.VMEM((1, C), jnp.int32),
            pltpu.VMEM((C, DEGW), jnp.float32),
            pltpu.VMEM((RCH, DEGW), jnp.float32),
            pltpu.VMEM_SHARED((NP, DEGW), jnp.float32),
            pltpu.SemaphoreType.DMA,
        ],
        compiler_params=pltpu.CompilerParams(use_tc_tiling_on_sc=False),
    )


RB = 2000


def _embed_body(h_ref, w_ref, b_ref, o_ref):
    o_ref[...] = (
        jnp.dot(h_ref[...], w_ref[...], preferred_element_type=jnp.float32)
        + b_ref[...]
    )


def _embed(h, W, b):
    return pl.pallas_call(
        _embed_body,
        grid=(N // RB,),
        in_specs=[
            pl.BlockSpec((RB, D), lambda i: (i, 0)),
            pl.BlockSpec((D, D), lambda i: (0, 0)),
            pl.BlockSpec((1, D), lambda i: (0, 0)),
        ],
        out_specs=pl.BlockSpec((RB, D), lambda i: (i, 0)),
        out_shape=jax.ShapeDtypeStruct((N, D), jnp.float32),
    )(h, W, b.reshape(1, D))


def _sage_update(x, p0, p1, invdeg, wt, wb, b):
    hn = (p0 + p1) * invdeg
    bundle = (
        jnp.dot(x, wt, preferred_element_type=jnp.float32)
        + jnp.dot(hn, wb, preferred_element_type=jnp.float32)
        + b
    )
    nrm = jnp.sqrt(jnp.sum(bundle * bundle, axis=1, keepdims=True))
    return x + jnp.maximum(bundle / jnp.maximum(nrm, 1e-12), 0.0)


def _layer_body(x_ref, p0_ref, p1_ref, iv_ref, wt_ref, wb_ref, b_ref, o_ref):
    o_ref[...] = _sage_update(
        x_ref[...], p0_ref[0], p1_ref[0], iv_ref[...][:, :1],
        wt_ref[...], wb_ref[...], b_ref[...])


def _layer(x, part, invdeg, W, b):
    return pl.pallas_call(
        _layer_body,
        grid=(N // RB,),
        in_specs=[
            pl.BlockSpec((RB, D), lambda i: (i, 0)),
            pl.BlockSpec((1, RB, D), lambda i: (0, i, 0)),
            pl.BlockSpec((1, RB, D), lambda i: (1, i, 0)),
            pl.BlockSpec((RB, DEGW), lambda i: (i, 0)),
            pl.BlockSpec((D, D), lambda i: (0, 0)),
            pl.BlockSpec((D, D), lambda i: (0, 0)),
            pl.BlockSpec((1, D), lambda i: (0, 0)),
        ],
        out_specs=pl.BlockSpec((RB, D), lambda i: (i, 0)),
        out_shape=jax.ShapeDtypeStruct((N, D), jnp.float32),
    )(x, part, part, invdeg, W[:D], W[D:], b.reshape(1, D))


def _final_body(x_ref, p0_ref, p1_ref, iv_ref, wt_ref, wb_ref, b_ref,
                prot_ref, wfc_ref, o_ref):
    x3 = _sage_update(
        x_ref[...], p0_ref[0], p1_ref[0], iv_ref[...][:, :1],
        wt_ref[...], wb_ref[...], b_ref[...])
    P = prot_ref[...]
    wfc = wfc_ref[...]
    y = jnp.zeros((), jnp.float32)
    for k in range(NPROT):
        diff = x3 - P[k:k + 1, :]
        d2 = jnp.sum(diff * diff, axis=1)
        sim = jnp.log((d2 + 1.0) / (d2 + 1e-12))
        y = y + jnp.max(sim) * wfc[0, k]
    o_ref[...] = jnp.full((1, 8, D), 1.0 / (1.0 + jnp.exp(-y)), jnp.float32)


def _final(x, part, invdeg, W, b, prot, wfc):
    return pl.pallas_call(
        _final_body,
        grid=(G,),
        in_specs=[
            pl.BlockSpec((NPG, D), lambda i: (i, 0)),
            pl.BlockSpec((1, NPG, D), lambda i: (0, i, 0)),
            pl.BlockSpec((1, NPG, D), lambda i: (1, i, 0)),
            pl.BlockSpec((NPG, DEGW), lambda i: (i, 0)),
            pl.BlockSpec((D, D), lambda i: (0, 0)),
            pl.BlockSpec((D, D), lambda i: (0, 0)),
            pl.BlockSpec((1, D), lambda i: (0, 0)),
            pl.BlockSpec((16, D), lambda i: (0, 0)),
            pl.BlockSpec((8, D), lambda i: (0, 0)),
        ],
        out_specs=pl.BlockSpec((1, 8, D), lambda i: (i, 0, 0)),
        out_shape=jax.ShapeDtypeStruct((G, 8, D), jnp.float32),
    )(x, part, part, invdeg, W[:D], W[D:], b.reshape(1, D), prot, wfc)


def kernel(h, e, edge_index, graph_ids, W_embed, b_embed,
           W0, b0, W1, b1, W2, b2, p_pos, p_neg, W_fc):
    packed = (edge_index[0] | (edge_index[1] << IDXSHIFT)).reshape(NW, NCH, C)

    x0 = _embed(h, W_embed, b_embed)
    dg = _make_deg()(packed)
    p0 = _make_agg()(x0, packed)
    invdeg = (1.0 / jnp.maximum(dg[0, :, :1] + dg[1, :, :1], 1.0)) * jnp.ones(
        (1, DEGW), jnp.float32)

    x1 = _layer(x0, p0, invdeg, W0, b0)
    p1 = _make_agg()(x1, packed)
    x2 = _layer(x1, p1, invdeg, W1, b1)
    p2 = _make_agg()(x2, packed)

    prot = jnp.zeros((16, D), jnp.float32).at[:NPROT].set(
        jnp.concatenate([p_pos, p_neg], axis=0))
    wfc = jnp.zeros((8, D), jnp.float32).at[0, :NPROT].set(W_fc[0])
    y = _final(x2, p2, invdeg, W2, b2, prot, wfc)
    return y[:, 0, 0]


# speedup vs baseline: 10.3550x; 1.0078x over previous
import functools

import jax
import jax.numpy as jnp
from jax import lax
from jax.experimental import pallas as pl
from jax.experimental.pallas import tpu as pltpu
from jax.experimental.pallas import tpu_sc as plsc

N = 10000
E = 320000
D = 128
G = 50
NPG = 200
NPROT = 10

NC = 2
NS = 16
NW = NC * NS
EPW = E // NW
C = 80
NCH = EPW // C
NP = 10240
RPS = NP // NS
RCH = 80
NZ = RPS // RCH
DEGW = 16
IDXSHIFT = 14


def _sc_mesh():
    return plsc.VectorSubcoreMesh(
        core_axis_name="c", subcore_axis_name="s",
        num_cores=NC, num_subcores=NS)


def _zero_fill(buf, nrows, width):
    zero16 = jnp.zeros((16,), jnp.float32)

    def _f(i, carry):
        for jj in range(width // 16):
            buf[i, pl.ds(jj * 16, 16)] = zero16
        return carry
    lax.fori_loop(0, nrows, _f, 0)


def _agg_body(x_hbm, packed_hbm, out,
              packv, srcv, dstv, rows, accum, sem0, sem1):
    cid = lax.axis_index("c")
    sid = lax.axis_index("s")
    w = cid * NS + sid
    base = sid * RPS

    _zero_fill(rows.at[0], RCH, D)
    for z in range(NZ):
        pltpu.sync_copy(rows.at[0], accum.at[pl.ds(base + z * RCH, RCH)])

    pltpu.sync_copy(packed_hbm.at[w], packv)

    def _unpack(j, b):
        for q in range(C // 16):
            v = packv[j, pl.ds(q * 16, 16)]
            srcv[b, pl.ds(q * 16, 16)] = v & ((1 << IDXSHIFT) - 1)
            dstv[b, pl.ds(q * 16, 16)] = v >> IDXSHIFT

    plsc.subcore_barrier()

    _unpack(0, 0)
    pltpu.async_copy(x_hbm.at[srcv.at[0]], rows.at[0], sem0)

    def _pair(t, carry):
        j0 = 2 * t
        _unpack(j0 + 1, 1)
        pltpu.async_copy(x_hbm.at[srcv.at[1]], rows.at[1], sem1)
        pltpu.make_async_copy(x_hbm.at[srcv.at[0]], rows.at[0], sem0).wait()
        pltpu.sync_copy(rows.at[0], accum.at[dstv.at[0]], add=True)
        _unpack(j0 + 2, 0)
        pltpu.async_copy(x_hbm.at[srcv.at[0]], rows.at[0], sem0)
        pltpu.make_async_copy(x_hbm.at[srcv.at[1]], rows.at[1], sem1).wait()
        pltpu.sync_copy(rows.at[1], accum.at[dstv.at[1]], add=True)
        return carry
    lax.fori_loop(0, (NCH - 1) // 2, _pair, 0)

    pltpu.make_async_copy(x_hbm.at[srcv.at[0]], rows.at[0], sem0).wait()
    pltpu.sync_copy(rows.at[0], accum.at[dstv.at[0]], add=True)

    plsc.subcore_barrier()

    sl = pl.ds(base, RPS)
    pltpu.sync_copy(accum.at[sl], out.at[cid, sl])


def _deg_body(packed_hbm, deg, packv, dstv, onesv, stage, degacc, sem0):
    cid = lax.axis_index("c")
    sid = lax.axis_index("s")
    w = cid * NS + sid
    base = sid * RPS

    _zero_fill(stage, RCH, DEGW)
    for z in range(NZ):
        pltpu.sync_copy(stage, degacc.at[pl.ds(base + z * RCH, RCH)])
    ones16 = jnp.ones((16,), jnp.float32)

    def _f(i, carry):
        onesv[i, pl.ds(0, DEGW)] = ones16
        return carry
    lax.fori_loop(0, C, _f, 0)
    pltpu.sync_copy(packed_hbm.at[w], packv)

    plsc.subcore_barrier()

    def _chunk(j, carry):
        for q in range(C // 16):
            dstv[0, pl.ds(q * 16, 16)] = (
                packv[j, pl.ds(q * 16, 16)] >> IDXSHIFT)
        pltpu.sync_copy(onesv, degacc.at[dstv.at[0]], add=True)
        return carry
    lax.fori_loop(0, NCH, _chunk, 0)

    plsc.subcore_barrier()

    sl = pl.ds(base, RPS)
    pltpu.sync_copy(degacc.at[sl], deg.at[cid, sl])


@functools.lru_cache(maxsize=None)
def _make_agg():
    return pl.kernel(
        _agg_body,
        out_type=jax.ShapeDtypeStruct((NC, NP, D), jnp.float32),
        mesh=_sc_mesh(),
        scratch_types=[
            pltpu.VMEM((NCH, C), jnp.int32),
            pltpu.VMEM((2, C), jnp.int32),
            pltpu.VMEM((2, C), jnp.int32),
            pltpu.VMEM((2, C, D), jnp.float32),
            pltpu.VMEM_SHARED((NP, D), jnp.float32),
            pltpu.SemaphoreType.DMA,
            pltpu.SemaphoreType.DMA,
        ],
        compiler_params=pltpu.CompilerParams(use_tc_tiling_on_sc=False),
    )


@functools.lru_cache(maxsize=None)
def _make_deg():
    return pl.kernel(
        _deg_body,
        out_type=jax.ShapeDtypeStruct((NC, NP, DEGW), jnp.float32),
        mesh=_sc_mesh(),
        scratch_types=[
            pltpu.VMEM((NCH, C), jnp.int32),
            pltpu.VMEM((1, C), jnp.int32),
            pltpu.VMEM((C, DEGW), jnp.float32),
            pltpu.VMEM((RCH, DEGW), jnp.float32),
            pltpu.VMEM_SHARED((NP, DEGW), jnp.float32),
            pltpu.SemaphoreType.DMA,
        ],
        compiler_params=pltpu.CompilerParams(use_tc_tiling_on_sc=False),
    )


RB = 2000


def _embed_body(h_ref, w_ref, b_ref, o_ref):
    o_ref[...] = (
        jnp.dot(h_ref[...], w_ref[...], preferred_element_type=jnp.float32)
        + b_ref[...]
    )


def _embed(h, W, b):
    return pl.pallas_call(
        _embed_body,
        grid=(N // RB,),
        in_specs=[
            pl.BlockSpec((RB, D), lambda i: (i, 0)),
            pl.BlockSpec((D, D), lambda i: (0, 0)),
            pl.BlockSpec((1, D), lambda i: (0, 0)),
        ],
        out_specs=pl.BlockSpec((RB, D), lambda i: (i, 0)),
        out_shape=jax.ShapeDtypeStruct((N, D), jnp.float32),
    )(h, W, b.reshape(1, D))


def _sage_update(x, p0, p1, invdeg, wt, wb, b):
    hn = (p0 + p1) * invdeg
    bundle = (
        jnp.dot(x, wt, preferred_element_type=jnp.float32)
        + jnp.dot(hn, wb, preferred_element_type=jnp.float32)
        + b
    )
    nrm = jnp.sqrt(jnp.sum(bundle * bundle, axis=1, keepdims=True))
    return x + jnp.maximum(bundle / jnp.maximum(nrm, 1e-12), 0.0)


def _inv_deg(d0, d1):
    return 1.0 / jnp.maximum(d0[:, :1] + d1[:, :1], 1.0)


def _layer_body(x_ref, p0_ref, p1_ref, d0_ref, d1_ref,
                wt_ref, wb_ref, b_ref, o_ref):
    o_ref[...] = _sage_update(
        x_ref[...], p0_ref[0], p1_ref[0], _inv_deg(d0_ref[0], d1_ref[0]),
        wt_ref[...], wb_ref[...], b_ref[...])


def _layer(x, part, dg, W, b):
    return pl.pallas_call(
        _layer_body,
        grid=(N // RB,),
        in_specs=[
            pl.BlockSpec((RB, D), lambda i: (i, 0)),
            pl.BlockSpec((1, RB, D), lambda i: (0, i, 0)),
            pl.BlockSpec((1, RB, D), lambda i: (1, i, 0)),
            pl.BlockSpec((1, RB, DEGW), lambda i: (0, i, 0)),
            pl.BlockSpec((1, RB, DEGW), lambda i: (1, i, 0)),
            pl.BlockSpec((D, D), lambda i: (0, 0)),
            pl.BlockSpec((D, D), lambda i: (0, 0)),
            pl.BlockSpec((1, D), lambda i: (0, 0)),
        ],
        out_specs=pl.BlockSpec((RB, D), lambda i: (i, 0)),
        out_shape=jax.ShapeDtypeStruct((N, D), jnp.float32),
    )(x, part, part, dg, dg, W[:D], W[D:], b.reshape(1, D))


def _final_body(x_ref, p0_ref, p1_ref, d0_ref, d1_ref, wt_ref, wb_ref, b_ref,
                pp_ref, pn_ref, wfc_ref, o_ref):
    x3 = _sage_update(
        x_ref[...], p0_ref[0], p1_ref[0], _inv_deg(d0_ref[0], d1_ref[0]),
        wt_ref[...], wb_ref[...], b_ref[...])
    wfc = wfc_ref[...]
    pp = pp_ref[...]
    pn = pn_ref[...]
    y = jnp.zeros((), jnp.float32)
    for k in range(NPROT):
        P = pp if k < NPROT // 2 else pn
        diff = x3 - P[k % (NPROT // 2)][None, :]
        d2 = jnp.sum(diff * diff, axis=1)
        sim = jnp.log((d2 + 1.0) / (d2 + 1e-12))
        y = y + jnp.max(sim) * wfc[0, k]
    o_ref[...] = jnp.full((1, 8, D), 1.0 / (1.0 + jnp.exp(-y)), jnp.float32)


def _final(x, part, dg, W, b, p_pos, p_neg, wfc):
    return pl.pallas_call(
        _final_body,
        grid=(G,),
        in_specs=[
            pl.BlockSpec((NPG, D), lambda i: (i, 0)),
            pl.BlockSpec((1, NPG, D), lambda i: (0, i, 0)),
            pl.BlockSpec((1, NPG, D), lambda i: (1, i, 0)),
            pl.BlockSpec((1, NPG, DEGW), lambda i: (0, i, 0)),
            pl.BlockSpec((1, NPG, DEGW), lambda i: (1, i, 0)),
            pl.BlockSpec((D, D), lambda i: (0, 0)),
            pl.BlockSpec((D, D), lambda i: (0, 0)),
            pl.BlockSpec((1, D), lambda i: (0, 0)),
            pl.BlockSpec((NPROT // 2, D), lambda i: (0, 0)),
            pl.BlockSpec((NPROT // 2, D), lambda i: (0, 0)),
            pl.BlockSpec((1, NPROT), lambda i: (0, 0)),
        ],
        out_specs=pl.BlockSpec((1, 8, D), lambda i: (i, 0, 0)),
        out_shape=jax.ShapeDtypeStruct((G, 8, D), jnp.float32),
    )(x, part, part, dg, dg, W[:D], W[D:], b.reshape(1, D), p_pos, p_neg, wfc)


def kernel(h, e, edge_index, graph_ids, W_embed, b_embed,
           W0, b0, W1, b1, W2, b2, p_pos, p_neg, W_fc):
    packed = (edge_index[0] | (edge_index[1] << IDXSHIFT)).reshape(NW, NCH, C)

    x0 = _embed(h, W_embed, b_embed)
    dg = _make_deg()(packed)
    p0 = _make_agg()(x0, packed)

    x1 = _layer(x0, p0, dg, W0, b0)
    p1 = _make_agg()(x1, packed)
    x2 = _layer(x1, p1, dg, W1, b1)
    p2 = _make_agg()(x2, packed)

    y = _final(x2, p2, dg, W2, b2, p_pos, p_neg, W_fc)
    return y[:, 0, 0]
